# baseline (speedup 1.0000x reference)
import functools

import jax
import jax.numpy as jnp
from jax import lax
from jax.experimental import pallas as pl
from jax.experimental.pallas import tpu as pltpu
from jax.experimental.pallas import tpu_sc as plsc

VDIM = 256
HALF = 128
NEXP = 16
BATCH = 1024
TOK = 2
OUT_DIM = 256
NTOK = BATCH * TOK
NC, NS = 2, 16
NW = NC * NS
TOK_PER_W = NTOK // NW


@functools.cache
def _make_gather():
    mesh = plsc.VectorSubcoreMesh(core_axis_name="c", subcore_axis_name="s")

    @functools.partial(
        pl.kernel,
        out_type=jax.ShapeDtypeStruct((NTOK, VDIM), jnp.float32),
        mesh=mesh,
        scratch_types=[
            pltpu.VMEM((TOK_PER_W,), jnp.int32),
            pltpu.VMEM((TOK_PER_W, VDIM), jnp.float32),
            pltpu.SemaphoreType.DMA,
        ],
    )
    def gather_rows(idx_hbm, table_hbm, out_hbm, idx_v, rows_v, sem):
        wid = lax.axis_index("s") * NC + lax.axis_index("c")
        base = wid * TOK_PER_W
        pltpu.sync_copy(idx_hbm.at[pl.ds(base, TOK_PER_W)], idx_v)
        pltpu.async_copy(table_hbm.at[idx_v], rows_v, sem).wait()
        pltpu.sync_copy(rows_v, out_hbm.at[pl.ds(base, TOK_PER_W)])

    return gather_rows


def _combine_body(cond_ref, v_ref, s_ref, l_ref, w_ref, out_ref):
    c = cond_ref[0, 0]
    v0 = v_ref[:, 0, :]
    v1 = v_ref[:, 1, :]
    v0h = v0[:, :HALF] * (1.0 - c) + v0[:, HALF:] * c
    v1h = v1[:, :HALF] * (1.0 - c) + v1[:, HALF:] * c
    s0 = s_ref[:, 0:1]
    s1 = s_ref[:, 1:2]
    l0 = l_ref[:, 0:1]
    l1 = l_ref[:, 1:2]
    w = w_ref[:, :HALF, :] * (1.0 - c) + w_ref[:, HALF:, :] * c
    wb = w.astype(jnp.bfloat16)
    acc = jnp.zeros((BATCH, OUT_DIM), jnp.float32)
    for e in range(NEXP):
        m0 = jnp.where(l0 == e, s0, 0.0)
        m1 = jnp.where(l1 == e, s1, 0.0)
        x = (v0h * m0 + v1h * m1).astype(jnp.bfloat16)
        acc = acc + jnp.dot(x, wb[e], preferred_element_type=jnp.float32)
    out_ref[...] = acc


_combine = pl.pallas_call(
    _combine_body,
    out_shape=jax.ShapeDtypeStruct((BATCH, OUT_DIM), jnp.float32),
    in_specs=[
        pl.BlockSpec(memory_space=pltpu.SMEM),
        pl.BlockSpec(),
        pl.BlockSpec(),
        pl.BlockSpec(),
        pl.BlockSpec(),
    ],
)


def kernel(indices, scores, W, label, index, weight):
    idx_flat = indices.reshape(-1).astype(jnp.int32)
    rows = _make_gather()(idx_flat, weight)
    cond = (jnp.asarray(index) == 1).astype(jnp.float32).reshape(1, 1)
    v = rows.reshape(BATCH, TOK, VDIM)
    return _combine(cond, v, scores, label.astype(jnp.int32), W)


# speedup vs baseline: 1.4829x; 1.4829x over previous
import functools

import jax
import jax.numpy as jnp
from jax import lax
from jax.experimental import pallas as pl
from jax.experimental.pallas import tpu as pltpu
from jax.experimental.pallas import tpu_sc as plsc

VDIM = 256
HALF = 128
NEXP = 16
BATCH = 1024
TOK = 2
OUT_DIM = 256
NTOK = BATCH * TOK
NC, NS = 2, 16
NW = NC * NS
TOK_PER_W = NTOK // NW


@functools.cache
def _make_gather():
    mesh = plsc.VectorSubcoreMesh(core_axis_name="c", subcore_axis_name="s")

    @functools.partial(
        pl.kernel,
        out_type=jax.ShapeDtypeStruct((NTOK, VDIM), jnp.float32),
        mesh=mesh,
        scratch_types=[
            pltpu.VMEM((TOK_PER_W,), jnp.int32),
            pltpu.VMEM((TOK_PER_W, VDIM), jnp.float32),
            pltpu.SemaphoreType.DMA,
        ],
    )
    def gather_rows(idx_hbm, table_hbm, out_hbm, idx_v, rows_v, sem):
        wid = lax.axis_index("s") * NC + lax.axis_index("c")
        base = wid * TOK_PER_W
        pltpu.sync_copy(idx_hbm.at[pl.ds(base, TOK_PER_W)], idx_v)
        pltpu.async_copy(table_hbm.at[idx_v], rows_v, sem).wait()
        pltpu.sync_copy(rows_v, out_hbm.at[pl.ds(base, TOK_PER_W)])

    return gather_rows


def _combine_body(cond_ref, v_ref, s_ref, l_ref, w_ref, out_ref):
    off = pl.multiple_of(cond_ref[0, 0] * HALF, HALF)
    v0h = v_ref[:, 0, pl.ds(off, HALF)]
    v1h = v_ref[:, 1, pl.ds(off, HALF)]
    wh = w_ref[:, pl.ds(off, HALF), :].reshape(NEXP * HALF, OUT_DIM)
    s0 = s_ref[:, 0:1]
    s1 = s_ref[:, 1:2]
    l0 = l_ref[:, 0:1]
    l1 = l_ref[:, 1:2]
    y0 = v0h * s0
    y1 = v1h * s1
    blocks = [
        jnp.where(l0 == e, y0, 0.0) + jnp.where(l1 == e, y1, 0.0)
        for e in range(NEXP)
    ]
    a = jnp.concatenate(blocks, axis=1)
    out_ref[...] = jnp.dot(a, wh, preferred_element_type=jnp.float32)


_combine = pl.pallas_call(
    _combine_body,
    out_shape=jax.ShapeDtypeStruct((BATCH, OUT_DIM), jnp.float32),
    in_specs=[
        pl.BlockSpec(memory_space=pltpu.SMEM),
        pl.BlockSpec(),
        pl.BlockSpec(),
        pl.BlockSpec(),
        pl.BlockSpec(),
    ],
)


def kernel(indices, scores, W, label, index, weight):
    idx_flat = indices.reshape(-1).astype(jnp.int32)
    rows = _make_gather()(idx_flat, weight)
    cond = (jnp.asarray(index) == 1).astype(jnp.int32).reshape(1, 1)
    v = rows.reshape(BATCH, TOK, VDIM)
    return _combine(cond, v, scores, label.astype(jnp.int32), W)


# speedup vs baseline: 1.4997x; 1.0113x over previous
import functools

import jax
import jax.numpy as jnp
from jax import lax
from jax.experimental import pallas as pl
from jax.experimental.pallas import tpu as pltpu
from jax.experimental.pallas import tpu_sc as plsc

VDIM = 256
HALF = 128
NEXP = 16
BATCH = 1024
TOK = 2
OUT_DIM = 256
NTOK = BATCH * TOK
NC, NS = 2, 16
NW = NC * NS
TOK_PER_W = NTOK // NW


@functools.cache
def _make_gather():
    mesh = plsc.VectorSubcoreMesh(core_axis_name="c", subcore_axis_name="s")

    @functools.partial(
        pl.kernel,
        out_type=jax.ShapeDtypeStruct((NTOK, VDIM), jnp.float32),
        mesh=mesh,
        scratch_types=[
            pltpu.VMEM((TOK_PER_W,), jnp.int32),
            pltpu.VMEM((TOK_PER_W, VDIM), jnp.float32),
            pltpu.SemaphoreType.DMA,
        ],
    )
    def gather_rows(idx_hbm, table_hbm, out_hbm, idx_v, rows_v, sem):
        wid = lax.axis_index("s") * NC + lax.axis_index("c")
        base = wid * TOK_PER_W
        pltpu.sync_copy(idx_hbm.at[pl.ds(base, TOK_PER_W)], idx_v)
        pltpu.async_copy(table_hbm.at[idx_v], rows_v, sem).wait()
        pltpu.sync_copy(rows_v, out_hbm.at[pl.ds(base, TOK_PER_W)])

    return gather_rows


def _combine_body(cond_ref, v_ref, s_ref, l_ref, w_ref, out_ref):
    off = pl.multiple_of(cond_ref[0, 0] * HALF, HALF)
    v0h = v_ref[:, 0, pl.ds(off, HALF)]
    v1h = v_ref[:, 1, pl.ds(off, HALF)]
    wh = w_ref[:, pl.ds(off, HALF), :].reshape(NEXP * HALF, OUT_DIM)
    s0 = s_ref[:, 0:1]
    s1 = s_ref[:, 1:2]
    l0 = l_ref[:, 0:1]
    l1 = l_ref[:, 1:2]
    y0 = v0h * s0
    y1 = v1h * s1
    blocks = [
        jnp.where(l0 == e, y0, 0.0) + jnp.where(l1 == e, y1, 0.0)
        for e in range(NEXP)
    ]
    a = jnp.concatenate(blocks, axis=1)
    out_ref[...] = jnp.dot(a, wh, preferred_element_type=jnp.float32)


_combine = pl.pallas_call(
    _combine_body,
    out_shape=jax.ShapeDtypeStruct((BATCH, OUT_DIM), jnp.float32),
    in_specs=[
        pl.BlockSpec(memory_space=pltpu.SMEM),
        pl.BlockSpec(),
        pl.BlockSpec(),
        pl.BlockSpec(),
        pl.BlockSpec(),
    ],
)


@functools.cache
def _make_gather1():
    mesh = plsc.VectorSubcoreMesh(core_axis_name="c", subcore_axis_name="s",
                                  num_cores=1)
    tpw = NTOK // NS

    @functools.partial(
        pl.kernel,
        out_type=jax.ShapeDtypeStruct((NTOK, VDIM), jnp.float32),
        mesh=mesh,
        scratch_types=[
            pltpu.VMEM((tpw,), jnp.int32),
            pltpu.VMEM((tpw, VDIM), jnp.float32),
            pltpu.SemaphoreType.DMA,
        ],
    )
    def gather_rows(idx_hbm, table_hbm, out_hbm, idx_v, rows_v, sem):
        wid = lax.axis_index("s")
        base = wid * tpw
        pltpu.sync_copy(idx_hbm.at[pl.ds(base, tpw)], idx_v)
        pltpu.async_copy(table_hbm.at[idx_v], rows_v, sem).wait()
        pltpu.sync_copy(rows_v, out_hbm.at[pl.ds(base, tpw)])

    return gather_rows


def kernel(indices, scores, W, label, index, weight):
    idx_flat = indices.reshape(-1).astype(jnp.int32)
    rows = _make_gather()(idx_flat, weight)
    cond = (jnp.asarray(index) == 1).astype(jnp.int32).reshape(1, 1)
    v = rows.reshape(BATCH, TOK, VDIM)
    return _combine(cond, v, scores, label.astype(jnp.int32), W)


# speedup vs baseline: 1.6915x; 1.1279x over previous
import functools

import jax
import jax.numpy as jnp
from jax import lax
from jax.experimental import pallas as pl
from jax.experimental.pallas import tpu as pltpu
from jax.experimental.pallas import tpu_sc as plsc

VDIM = 256
HALF = 128
NEXP = 16
BATCH = 1024
TOK = 2
OUT_DIM = 256
NTOK = BATCH * TOK
NC, NS = 2, 16
NW = NC * NS
TOK_PER_W = NTOK // NW


@functools.cache
def _make_gather():
    mesh = plsc.VectorSubcoreMesh(core_axis_name="c", subcore_axis_name="s")

    @functools.partial(
        pl.kernel,
        out_type=jax.ShapeDtypeStruct((NTOK, VDIM), jnp.float32),
        mesh=mesh,
        scratch_types=[
            pltpu.VMEM((TOK_PER_W,), jnp.int32),
            pltpu.VMEM((TOK_PER_W, VDIM), jnp.float32),
            pltpu.SemaphoreType.DMA,
        ],
    )
    def gather_rows(idx_hbm, table_hbm, out_hbm, idx_v, rows_v, sem):
        wid = lax.axis_index("s") * NC + lax.axis_index("c")
        base = wid * TOK_PER_W
        pltpu.sync_copy(idx_hbm.at[pl.ds(base, TOK_PER_W)], idx_v)
        pltpu.async_copy(table_hbm.at[idx_v], rows_v, sem).wait()
        pltpu.sync_copy(rows_v, out_hbm.at[pl.ds(base, TOK_PER_W)])

    return gather_rows


def _combine_body(cond_ref, v_ref, s_ref, l_ref, w_ref, out_ref):
    off = pl.multiple_of(cond_ref[0, 0] * HALF, HALF)
    v0h = v_ref[:BATCH, pl.ds(off, HALF)]
    v1h = v_ref[BATCH:, pl.ds(off, HALF)]
    wh = w_ref[:, pl.ds(off, HALF), :].reshape(NEXP * HALF, OUT_DIM)
    s0 = s_ref[:, 0:1]
    s1 = s_ref[:, 1:2]
    l0 = l_ref[:, 0:1]
    l1 = l_ref[:, 1:2]
    y0 = v0h * s0
    y1 = v1h * s1
    blocks = [
        jnp.where(l0 == e, y0, 0.0) + jnp.where(l1 == e, y1, 0.0)
        for e in range(NEXP)
    ]
    a = jnp.concatenate(blocks, axis=1)
    out_ref[...] = jnp.dot(a, wh, preferred_element_type=jnp.float32)


_combine = pl.pallas_call(
    _combine_body,
    out_shape=jax.ShapeDtypeStruct((BATCH, OUT_DIM), jnp.float32),
    in_specs=[
        pl.BlockSpec(memory_space=pltpu.SMEM),
        pl.BlockSpec(),
        pl.BlockSpec(),
        pl.BlockSpec(),
        pl.BlockSpec(),
    ],
)


@functools.cache
def _make_gather1():
    mesh = plsc.VectorSubcoreMesh(core_axis_name="c", subcore_axis_name="s",
                                  num_cores=1)
    tpw = NTOK // NS

    @functools.partial(
        pl.kernel,
        out_type=jax.ShapeDtypeStruct((NTOK, VDIM), jnp.float32),
        mesh=mesh,
        scratch_types=[
            pltpu.VMEM((tpw,), jnp.int32),
            pltpu.VMEM((tpw, VDIM), jnp.float32),
            pltpu.SemaphoreType.DMA,
        ],
    )
    def gather_rows(idx_hbm, table_hbm, out_hbm, idx_v, rows_v, sem):
        wid = lax.axis_index("s")
        base = wid * tpw
        pltpu.sync_copy(idx_hbm.at[pl.ds(base, tpw)], idx_v)
        pltpu.async_copy(table_hbm.at[idx_v], rows_v, sem).wait()
        pltpu.sync_copy(rows_v, out_hbm.at[pl.ds(base, tpw)])

    return gather_rows


def kernel(indices, scores, W, label, index, weight):
    idx_flat = indices.T.reshape(-1).astype(jnp.int32)
    rows = _make_gather()(idx_flat, weight)
    cond = (jnp.asarray(index) == 1).astype(jnp.int32).reshape(1, 1)
    return _combine(cond, rows, scores, label.astype(jnp.int32), W)
